# baseline (speedup 1.0000x reference)
import dataclasses
import functools

import jax
import jax.numpy as jnp
from jax import lax
from jax.experimental import pallas as pl
from jax.experimental.pallas import tpu as pltpu
from jax.experimental.pallas import tpu_sc as plsc

N = 10000
E = 320000
D = 128
DELTA = 2.5
EPS = 1e-5

NC = 2
NS = 16
NW = NC * NS
EPW = E // NW
G = 80
NCHUNK = EPW // G
FPW = D // NW
D2 = D // 2
FPP = D2 // NW
NEGW = int(0xFF7FFF7F) - (1 << 32)
CB = 0
CH = 2000
NCH = E // CH
RPS = 624
RTAIL = N - NS * RPS
NEG = -3.0e38

_sc_mesh = plsc.VectorSubcoreMesh(core_axis_name="c", subcore_axis_name="s")

_sc_params = pltpu.CompilerParams()
if "needs_layout_passes" in pltpu.CompilerParams.__dataclass_fields__:
    _sc_params = dataclasses.replace(_sc_params, needs_layout_passes=False)


def _pre_body(ndata_ref, mw_ref, p_ref, q_ref, ptp_ref):
    x = ndata_ref[...]
    mw = mw_ref[...]
    p = jnp.dot(x, mw[:D], preferred_element_type=jnp.float32)
    p_ref[...] = p
    q_ref[...] = jnp.dot(x, mw[D:], preferred_element_type=jnp.float32)
    pb = p.T.astype(jnp.bfloat16)
    lo = lax.bitcast_convert_type(pb[:D2], jnp.uint16).astype(jnp.uint32)
    hi = lax.bitcast_convert_type(pb[D2:], jnp.uint16).astype(jnp.uint32)
    ptp_ref[...] = lax.bitcast_convert_type(lo | (hi << 16), jnp.int32)


def _pre(ndata, M_w):
    return pl.pallas_call(
        _pre_body,
        out_shape=(
            jax.ShapeDtypeStruct((N, D), jnp.float32),
            jax.ShapeDtypeStruct((N, D), jnp.float32),
            jax.ShapeDtypeStruct((D2, N), jnp.int32),
        ),
    )(ndata, M_w)


def _post_deg_body(deg_ref, dinv_ref, amp_ref):
    d = deg_ref[...] + 1.0
    dinv_ref[...] = jnp.broadcast_to(1.0 / d, (N, D))
    amp_ref[...] = jnp.broadcast_to(jnp.log1p(d) / DELTA, (N, D))


def _post_mean_body(p_ref, q_ref, s_ref, mb_ref, dinv_ref, amp_ref,
                    mean_ref, amean_ref):
    qb = q_ref[...] + mb_ref[...][None, :]
    mean = (s_ref[0] + s_ref[1] + p_ref[...]) * dinv_ref[...] + qb
    mean_ref[...] = mean
    amean_ref[...] = amp_ref[...] * mean


def _post_max_body(p_ref, q_ref, mxt_ref, mb_ref, amp_ref, mx_ref, amx_ref):
    qb = q_ref[...] + mb_ref[...][None, :]
    w = mxt_ref[...]
    lo = lax.bitcast_convert_type(lax.shift_left(w, 16), jnp.float32)
    hi = lax.bitcast_convert_type(
        lax.bitwise_and(w, jnp.int32(-65536)), jnp.float32)
    mxt = jnp.concatenate([lo, hi], axis=0)
    mx = jnp.maximum(mxt.T, p_ref[...]) + qb
    mx_ref[...] = mx
    amx_ref[...] = amp_ref[...] * mx


def _post_b_body(ndata_ref, mean_ref, mx_ref, amean_ref, amx_ref,
                 uw_ref, ub_ref, g_ref, b_ref, mixw_ref, mixb_ref, out_ref):
    ndata = ndata_ref[...]
    uw = uw_ref[...]
    h = (
        jnp.dot(ndata, uw[0 * D:1 * D], preferred_element_type=jnp.float32)
        + jnp.dot(mean_ref[...], uw[1 * D:2 * D] + uw[3 * D:4 * D],
                  preferred_element_type=jnp.float32)
        + jnp.dot(mx_ref[...], uw[2 * D:3 * D], preferred_element_type=jnp.float32)
        + jnp.dot(amean_ref[...], uw[4 * D:5 * D] + uw[6 * D:7 * D],
                  preferred_element_type=jnp.float32)
        + jnp.dot(amx_ref[...], uw[5 * D:6 * D], preferred_element_type=jnp.float32)
        + ub_ref[...][None, :]
    )
    mu = jnp.mean(h, axis=0, keepdims=True)
    var = jnp.mean((h - mu) ** 2, axis=0, keepdims=True)
    h = (h - mu) * lax.rsqrt(var + EPS) * g_ref[...][None, :] + b_ref[...][None, :]
    h = jnp.dot(h, mixw_ref[...], preferred_element_type=jnp.float32) + mixb_ref[...][None, :]
    h = jnp.where(h >= 0, h, 0.01 * h)
    out_ref[...] = h + ndata


def _post(ndata, p, q, sums, mxT, deg2d, M_b, U_w, U_b, bn_gamma, bn_beta,
          mix_w, mix_b):
    dinv, amp = pl.pallas_call(
        _post_deg_body,
        out_shape=(jax.ShapeDtypeStruct((N, D), jnp.float32),) * 2,
    )(deg2d)
    mean, amean = pl.pallas_call(
        _post_mean_body,
        out_shape=(jax.ShapeDtypeStruct((N, D), jnp.float32),) * 2,
    )(p, q, sums, M_b, dinv, amp)
    mx, amx = pl.pallas_call(
        _post_max_body,
        out_shape=(jax.ShapeDtypeStruct((N, D), jnp.float32),) * 2,
    )(p, q, mxT, M_b, amp)
    return pl.pallas_call(
        _post_b_body,
        out_shape=jax.ShapeDtypeStruct((N, D), jnp.float32),
    )(ndata, mean, mx, amean, amx, U_w, U_b, bn_gamma, bn_beta, mix_w, mix_b)


def _sc_sum_body(p_hbm, src_hbm, dst_hbm, zeros_hbm, sum_out,
                 src_v, dst_v, rows_v, sum_sh):
    c = lax.axis_index("c")
    s = lax.axis_index("s")
    wid = s * NC + c
    pltpu.sync_copy(zeros_hbm.at[pl.ds(s * RPS, RPS)],
                    sum_sh.at[pl.ds(s * RPS, RPS)])

    @pl.when(s == 0)
    def _ztail():
        pltpu.sync_copy(zeros_hbm.at[pl.ds(NS * RPS, RTAIL)],
                        sum_sh.at[pl.ds(NS * RPS, RTAIL)])

    plsc.subcore_barrier()

    base = wid * EPW

    @pl.loop(0, NCHUNK)
    def _chunk(ci):
        off = base + ci * G
        pltpu.sync_copy(src_hbm.at[pl.ds(off, G)], src_v)
        pltpu.sync_copy(dst_hbm.at[pl.ds(off, G)], dst_v)
        pltpu.sync_copy(p_hbm.at[src_v], rows_v)
        pltpu.sync_copy(rows_v, sum_sh.at[dst_v], add=True)

    plsc.subcore_barrier()
    pltpu.sync_copy(sum_sh.at[pl.ds(s * RPS, RPS)],
                    sum_out.at[c, pl.ds(s * RPS, RPS)])

    @pl.when(s == 0)
    def _dtail():
        pltpu.sync_copy(sum_sh.at[pl.ds(NS * RPS, RTAIL)],
                        sum_out.at[c, pl.ds(NS * RPS, RTAIL)])


def _sc_sum(p, src, dst, zeros):
    return pl.kernel(
        _sc_sum_body,
        out_type=jax.ShapeDtypeStruct((NC, N, D), jnp.float32),
        mesh=_sc_mesh,
        scratch_types=[
            pltpu.VMEM((G,), jnp.int32),
            pltpu.VMEM((G,), jnp.int32),
            pltpu.VMEM((G, D), jnp.float32),
            pltpu.VMEM_SHARED((N, D), jnp.float32),
        ],
    )(p, src, dst, zeros)


def _sc_max_body(ptf_hbm, src_hbm, dst_hbm, mxt_out, deg_out,
                 pt_v, acc_v, deg_v, src_v, dst_v, src_w, dst_w, sem0, sem1):
    c = lax.axis_index("c")
    s = lax.axis_index("s")
    wid = s * NC + c
    fbase = wid * FPP
    pltpu.sync_copy(ptf_hbm.at[pl.ds(fbase * N, FPP * N)], pt_v)

    neg = jnp.full((16,), NEGW, jnp.int32)
    zero = jnp.zeros((16,), jnp.float32)
    one = jnp.ones((16,), jnp.float32)

    @pl.loop(0, FPP * N, step=16)
    def _init(i):
        acc_v[pl.ds(i, 16)] = neg

    @pl.loop(0, N, step=16)
    def _dinit(i):
        deg_v[pl.ds(i, 16)] = zero

    def _start(ci, sbuf, dbuf, sem):
        pltpu.async_copy(src_hbm.at[pl.ds(ci * CH, CH)], sbuf, sem)
        pltpu.async_copy(dst_hbm.at[pl.ds(ci * CH, CH)], dbuf, sem)

    def _wait(sbuf, dbuf, sem):
        pltpu.make_async_copy(src_hbm.at[pl.ds(0, CH)], sbuf, sem).wait()
        pltpu.make_async_copy(dst_hbm.at[pl.ds(0, CH)], dbuf, sem).wait()

    def _process(src_v, dst_v):
        @pl.loop(0, CH, step=16)
        def _group(e):
            s16 = src_v[pl.ds(e, 16)]
            d16 = dst_v[pl.ds(e, 16)]
            counts, _ = plsc.scan_count(d16)
            ndup = plsc.all_reduce_population_count(counts != CB)
            vals = [
                plsc.bitcast(plsc.load_gather(pt_v, [s16 + (f * N)]),
                             jnp.bfloat16)
                for f in range(FPP)
            ]
            didx = [d16 + (f * N) for f in range(FPP)]

            def one_round(m):
                for f in range(FPP):
                    old = plsc.bitcast(plsc.load_gather(acc_v, [didx[f]]),
                                       jnp.bfloat16)
                    new = plsc.bitcast(jnp.maximum(old, vals[f]), jnp.int32)
                    plsc.store_scatter(acc_v, [didx[f]], new, mask=m)

                @pl.when(wid == 0)
                def _deg():
                    plsc.addupdate_scatter(deg_v, [d16], one, mask=m)

            one_round(counts == CB)

            @pl.when(ndup[0] > 0)
            def _rare():
                @pl.loop(1, 16)
                def _round(r):
                    one_round(counts == CB + r)

    _start(0, src_v, dst_v, sem0)

    @pl.loop(0, NCH, step=2)
    def _chunk(ci):
        _start(ci + 1, src_w, dst_w, sem1)
        _wait(src_v, dst_v, sem0)
        _process(src_v, dst_v)

        @pl.when(ci + 2 < NCH)
        def _pf():
            _start(ci + 2, src_v, dst_v, sem0)

        _wait(src_w, dst_w, sem1)
        _process(src_w, dst_w)

    pltpu.sync_copy(acc_v, mxt_out.at[pl.ds(fbase * N, FPP * N)])

    @pl.when(wid == 0)
    def _ddump():
        pltpu.sync_copy(deg_v, deg_out)


def _sc_max(ptf, src, dst):
    return pl.kernel(
        _sc_max_body,
        out_type=(
            jax.ShapeDtypeStruct((D2 * N,), jnp.int32),
            jax.ShapeDtypeStruct((N,), jnp.float32),
        ),
        mesh=_sc_mesh,
        compiler_params=_sc_params,
        scratch_types=[
            pltpu.VMEM((FPP * N,), jnp.int32),
            pltpu.VMEM((FPP * N,), jnp.int32),
            pltpu.VMEM((N,), jnp.float32),
            pltpu.VMEM((CH,), jnp.int32),
            pltpu.VMEM((CH,), jnp.int32),
            pltpu.VMEM((CH,), jnp.int32),
            pltpu.VMEM((CH,), jnp.int32),
            pltpu.SemaphoreType.DMA,
            pltpu.SemaphoreType.DMA,
        ],
    )(ptf, src, dst)


def kernel(ndata, edge_index, M_w, M_b, U_w, U_b, bn_gamma, bn_beta, mix_w,
           mix_b):
    src = edge_index[0]
    dst = edge_index[1]
    p, q, ptp = _pre(ndata, M_w)
    zeros = jnp.zeros((N, D), jnp.float32)
    sums = _sc_sum(p, src, dst, zeros)
    mxtf, deg = _sc_max(ptp.reshape(D2 * N), src, dst)
    return _post(ndata, p, q, sums, mxtf.reshape(D2, N), deg.reshape(N, 1),
                 M_b, U_w, U_b, bn_gamma, bn_beta, mix_w, mix_b)


# speedup vs baseline: 3.5358x; 3.5358x over previous
import dataclasses

import jax
import jax.numpy as jnp
from jax import lax
from jax.experimental import pallas as pl
from jax.experimental.pallas import tpu as pltpu
from jax.experimental.pallas import tpu_sc as plsc

N = 10000
E = 320000
D = 128
DELTA = 2.5
EPS = 1e-5

NC = 2
NS = 16
NW = NC * NS
EPW = E // NW
G = 80
NCHUNK = EPW // G
FPW = D // NW
D2 = D // 2
FPP = D2 // NW
NEGW = int(0xFF7FFF7F) - (1 << 32)
CH = 2000
NCH = E // CH
RPS = 624
RTAIL = N - NS * RPS

_sc_mesh = plsc.VectorSubcoreMesh(core_axis_name="c", subcore_axis_name="s")

_sc_params = pltpu.CompilerParams()
if "needs_layout_passes" in pltpu.CompilerParams.__dataclass_fields__:
    _sc_params = dataclasses.replace(_sc_params, needs_layout_passes=False)


def _pre_body(ndata_ref, mw_ref, p_ref, q_ref, ptp_ref):
    x = ndata_ref[...]
    mw = mw_ref[...]
    p = jnp.dot(x, mw[:D], preferred_element_type=jnp.float32)
    p_ref[...] = p
    q_ref[...] = jnp.dot(x, mw[D:], preferred_element_type=jnp.float32)
    pb = p.T.astype(jnp.bfloat16)
    lo = lax.bitcast_convert_type(pb[:D2], jnp.uint16).astype(jnp.uint32)
    hi = lax.bitcast_convert_type(pb[D2:], jnp.uint16).astype(jnp.uint32)
    ptp_ref[...] = lax.bitcast_convert_type(lo | (hi << 16), jnp.int32)


def _pre(ndata, M_w):
    return pl.pallas_call(
        _pre_body,
        out_shape=(
            jax.ShapeDtypeStruct((N, D), jnp.float32),
            jax.ShapeDtypeStruct((N, D), jnp.float32),
            jax.ShapeDtypeStruct((D2, N), jnp.int32),
        ),
    )(ndata, M_w)


def _post_deg_body(deg_ref, dinv_ref, amp_ref):
    d = deg_ref[...] + 1.0
    dinv_ref[...] = jnp.broadcast_to(1.0 / d, (N, D))
    amp_ref[...] = jnp.broadcast_to(jnp.log1p(d) / DELTA, (N, D))


def _post_mean_body(p_ref, q_ref, s_ref, mb_ref, dinv_ref, amp_ref,
                    mean_ref, amean_ref):
    qb = q_ref[...] + mb_ref[...][None, :]
    mean = (s_ref[0] + s_ref[1] + p_ref[...]) * dinv_ref[...] + qb
    mean_ref[...] = mean
    amean_ref[...] = amp_ref[...] * mean


def _post_max_body(p_ref, q_ref, mxt_ref, mb_ref, amp_ref, mx_ref, amx_ref):
    qb = q_ref[...] + mb_ref[...][None, :]
    w = mxt_ref[...]
    lo = lax.bitcast_convert_type(lax.shift_left(w, 16), jnp.float32)
    hi = lax.bitcast_convert_type(
        lax.bitwise_and(w, jnp.int32(-65536)), jnp.float32)
    mxt = jnp.concatenate([lo, hi], axis=0)
    mx = jnp.maximum(mxt.T, p_ref[...]) + qb
    mx_ref[...] = mx
    amx_ref[...] = amp_ref[...] * mx


def _post_b_body(ndata_ref, mean_ref, mx_ref, amean_ref, amx_ref,
                 uw_ref, ub_ref, g_ref, b_ref, mixw_ref, mixb_ref, out_ref):
    ndata = ndata_ref[...]
    uw = uw_ref[...]
    h = (
        jnp.dot(ndata, uw[0 * D:1 * D], preferred_element_type=jnp.float32)
        + jnp.dot(mean_ref[...], uw[1 * D:2 * D] + uw[3 * D:4 * D],
                  preferred_element_type=jnp.float32)
        + jnp.dot(mx_ref[...], uw[2 * D:3 * D], preferred_element_type=jnp.float32)
        + jnp.dot(amean_ref[...], uw[4 * D:5 * D] + uw[6 * D:7 * D],
                  preferred_element_type=jnp.float32)
        + jnp.dot(amx_ref[...], uw[5 * D:6 * D], preferred_element_type=jnp.float32)
        + ub_ref[...][None, :]
    )
    mu = jnp.mean(h, axis=0, keepdims=True)
    var = jnp.mean((h - mu) ** 2, axis=0, keepdims=True)
    h = (h - mu) * lax.rsqrt(var + EPS) * g_ref[...][None, :] + b_ref[...][None, :]
    h = jnp.dot(h, mixw_ref[...], preferred_element_type=jnp.float32) + mixb_ref[...][None, :]
    h = jnp.where(h >= 0, h, 0.01 * h)
    out_ref[...] = h + ndata


def _post(ndata, p, q, sums, mxT, deg2d, M_b, U_w, U_b, bn_gamma, bn_beta,
          mix_w, mix_b):
    dinv, amp = pl.pallas_call(
        _post_deg_body,
        out_shape=(jax.ShapeDtypeStruct((N, D), jnp.float32),) * 2,
    )(deg2d)
    mean, amean = pl.pallas_call(
        _post_mean_body,
        out_shape=(jax.ShapeDtypeStruct((N, D), jnp.float32),) * 2,
    )(p, q, sums, M_b, dinv, amp)
    mx, amx = pl.pallas_call(
        _post_max_body,
        out_shape=(jax.ShapeDtypeStruct((N, D), jnp.float32),) * 2,
    )(p, q, mxT, M_b, amp)
    return pl.pallas_call(
        _post_b_body,
        out_shape=jax.ShapeDtypeStruct((N, D), jnp.float32),
    )(ndata, mean, mx, amean, amx, U_w, U_b, bn_gamma, bn_beta, mix_w, mix_b)


def _sc_sum_body(p_hbm, src_hbm, dst_hbm, zeros_hbm, sum_out,
                 src_v, dst_v, rows_v, sum_sh):
    c = lax.axis_index("c")
    s = lax.axis_index("s")
    wid = s * NC + c
    pltpu.sync_copy(zeros_hbm.at[pl.ds(s * RPS, RPS)],
                    sum_sh.at[pl.ds(s * RPS, RPS)])

    @pl.when(s == 0)
    def _ztail():
        pltpu.sync_copy(zeros_hbm.at[pl.ds(NS * RPS, RTAIL)],
                        sum_sh.at[pl.ds(NS * RPS, RTAIL)])

    plsc.subcore_barrier()

    base = wid * EPW

    @pl.loop(0, NCHUNK)
    def _chunk(ci):
        off = base + ci * G
        pltpu.sync_copy(src_hbm.at[pl.ds(off, G)], src_v)
        pltpu.sync_copy(dst_hbm.at[pl.ds(off, G)], dst_v)
        pltpu.sync_copy(p_hbm.at[src_v], rows_v)
        pltpu.sync_copy(rows_v, sum_sh.at[dst_v], add=True)

    plsc.subcore_barrier()
    pltpu.sync_copy(sum_sh.at[pl.ds(s * RPS, RPS)],
                    sum_out.at[c, pl.ds(s * RPS, RPS)])

    @pl.when(s == 0)
    def _dtail():
        pltpu.sync_copy(sum_sh.at[pl.ds(NS * RPS, RTAIL)],
                        sum_out.at[c, pl.ds(NS * RPS, RTAIL)])


def _sc_sum(p, src, dst, zeros):
    return pl.kernel(
        _sc_sum_body,
        out_type=jax.ShapeDtypeStruct((NC, N, D), jnp.float32),
        mesh=_sc_mesh,
        scratch_types=[
            pltpu.VMEM((G,), jnp.int32),
            pltpu.VMEM((G,), jnp.int32),
            pltpu.VMEM((G, D), jnp.float32),
            pltpu.VMEM_SHARED((N, D), jnp.float32),
        ],
    )(p, src, dst, zeros)


def _sc_max_body(ptf_hbm, src_hbm, dst_hbm, mxt_out, deg_out,
                 pt_v, acc_v, deg_v, src_v, dst_v, src_w, dst_w, sem0, sem1):
    c = lax.axis_index("c")
    s = lax.axis_index("s")
    wid = s * NC + c
    fbase = wid * FPP
    pltpu.sync_copy(ptf_hbm.at[pl.ds(fbase * N, FPP * N)], pt_v)

    neg = jnp.full((16,), NEGW, jnp.int32)
    zero = jnp.zeros((16,), jnp.float32)
    one = jnp.ones((16,), jnp.float32)

    @pl.loop(0, FPP * N, step=16)
    def _init(i):
        acc_v[pl.ds(i, 16)] = neg

    @pl.loop(0, N, step=16)
    def _dinit(i):
        deg_v[pl.ds(i, 16)] = zero

    def _start(ci, sbuf, dbuf, sem):
        pltpu.async_copy(src_hbm.at[pl.ds(ci * CH, CH)], sbuf, sem)
        pltpu.async_copy(dst_hbm.at[pl.ds(ci * CH, CH)], dbuf, sem)

    def _wait(sbuf, dbuf, sem):
        pltpu.make_async_copy(src_hbm.at[pl.ds(0, CH)], sbuf, sem).wait()
        pltpu.make_async_copy(dst_hbm.at[pl.ds(0, CH)], dbuf, sem).wait()

    def _process(src_v, dst_v):
        @pl.loop(0, CH, step=16)
        def _group(e):
            s16 = src_v[pl.ds(e, 16)]
            d16 = dst_v[pl.ds(e, 16)]
            counts, _ = plsc.scan_count(d16)
            cbase = jnp.min(counts)
            cmax = jnp.max(counts)
            vals = [
                plsc.bitcast(plsc.load_gather(pt_v, [s16 + (f * N)]),
                             jnp.bfloat16)
                for f in range(FPP)
            ]
            didx = [d16 + (f * N) for f in range(FPP)]

            def one_round(m):
                for f in range(FPP):
                    old = plsc.bitcast(plsc.load_gather(acc_v, [didx[f]]),
                                       jnp.bfloat16)
                    new = plsc.bitcast(jnp.maximum(old, vals[f]), jnp.int32)
                    plsc.store_scatter(acc_v, [didx[f]], new, mask=m)

                @pl.when(wid == 0)
                def _deg():
                    plsc.addupdate_scatter(deg_v, [d16], one, mask=m)

            one_round(counts == cbase)

            @pl.when(cmax > cbase)
            def _rare():
                @pl.loop(1, 16)
                def _round(r):
                    one_round(counts == cbase + r)

    _start(0, src_v, dst_v, sem0)

    @pl.loop(0, NCH, step=2)
    def _chunk(ci):
        _start(ci + 1, src_w, dst_w, sem1)
        _wait(src_v, dst_v, sem0)
        _process(src_v, dst_v)

        @pl.when(ci + 2 < NCH)
        def _pf():
            _start(ci + 2, src_v, dst_v, sem0)

        _wait(src_w, dst_w, sem1)
        _process(src_w, dst_w)

    pltpu.sync_copy(acc_v, mxt_out.at[pl.ds(fbase * N, FPP * N)])

    @pl.when(wid == 0)
    def _ddump():
        pltpu.sync_copy(deg_v, deg_out)


def _sc_max(ptf, src, dst):
    return pl.kernel(
        _sc_max_body,
        out_type=(
            jax.ShapeDtypeStruct((D2 * N,), jnp.int32),
            jax.ShapeDtypeStruct((N,), jnp.float32),
        ),
        mesh=_sc_mesh,
        compiler_params=_sc_params,
        scratch_types=[
            pltpu.VMEM((FPP * N,), jnp.int32),
            pltpu.VMEM((FPP * N,), jnp.int32),
            pltpu.VMEM((N,), jnp.float32),
            pltpu.VMEM((CH,), jnp.int32),
            pltpu.VMEM((CH,), jnp.int32),
            pltpu.VMEM((CH,), jnp.int32),
            pltpu.VMEM((CH,), jnp.int32),
            pltpu.SemaphoreType.DMA,
            pltpu.SemaphoreType.DMA,
        ],
    )(ptf, src, dst)


def kernel(ndata, edge_index, M_w, M_b, U_w, U_b, bn_gamma, bn_beta, mix_w,
           mix_b):
    src = edge_index[0]
    dst = edge_index[1]
    p, q, ptp = _pre(ndata, M_w)
    zeros = jnp.zeros((N, D), jnp.float32)
    sums = _sc_sum(p, src, dst, zeros)
    mxtf, deg = _sc_max(ptp.reshape(D2 * N), src, dst)
    return _post(ndata, p, q, sums, mxtf.reshape(D2, N), deg.reshape(N, 1),
                 M_b, U_w, U_b, bn_gamma, bn_beta, mix_w, mix_b)
